# Initial kernel scaffold; baseline (speedup 1.0000x reference)
import jax
import jax.numpy as jnp
from jax.experimental import pallas as pl


def kernel(x, edge_index, edge_attr, W1, b1, W2, b2, gamma, beta):
    raise NotImplementedError("write your pallas kernel here")


# speedup vs baseline: 4.3801x; 4.3801x over previous
import functools

import jax
import jax.numpy as jnp
from jax import lax
from jax.experimental import pallas as pl
from jax.experimental.pallas import tpu as pltpu
from jax.experimental.pallas import tpu_sc as plsc

N = 10000
E = 320000
H = 128
NC = 2
NS = 16
NW = NC * NS
EPW = E // NW
CH = 80
NCHUNK = EPW // CH
NP = 10240
RPT = NP // NS
ZR = 128


def _sc_scatter_body(ea_hbm, idx_hbm, out_hbm, idx_v, buf_v, zbuf_v, acc_sh):
    c = lax.axis_index("c")
    s = lax.axis_index("s")
    wid = s * NC + c

    def zstore(i, _):
        zbuf_v[i // 8, pl.ds((i % 8) * 16, 16)] = jnp.zeros((16,), jnp.float32)
        return 0
    lax.fori_loop(0, ZR * 8, zstore, 0)
    for j in range(RPT // ZR):
        pltpu.sync_copy(zbuf_v, acc_sh.at[pl.ds(s * RPT + j * ZR, ZR), :])
    plsc.subcore_barrier()

    pltpu.sync_copy(idx_hbm.at[wid], idx_v)
    base = wid * EPW

    def chunk_body(ci, _):
        pltpu.sync_copy(ea_hbm.at[pl.ds(base + ci * CH, CH), :], buf_v)
        pltpu.sync_copy(buf_v, acc_sh.at[idx_v.at[ci]], add=True)
        return 0
    lax.fori_loop(0, NCHUNK, chunk_body, 0)
    plsc.subcore_barrier()

    pltpu.sync_copy(acc_sh.at[pl.ds(s * RPT, RPT), :],
                    out_hbm.at[c, pl.ds(s * RPT, RPT), :])


@functools.partial(
    pl.kernel,
    out_type=jax.ShapeDtypeStruct((NC, NP, H), jnp.float32),
    mesh=plsc.VectorSubcoreMesh(core_axis_name="c", subcore_axis_name="s"),
    scratch_types=[
        pltpu.VMEM((NCHUNK, CH), jnp.int32),
        pltpu.VMEM((CH, H), jnp.float32),
        pltpu.VMEM((ZR, H), jnp.float32),
        pltpu.VMEM_SHARED((NP, H), jnp.float32),
    ],
)
def _sc_scatter(ea_hbm, idx_hbm, out_hbm, idx_v, buf_v, zbuf_v, acc_sh):
    _sc_scatter_body(ea_hbm, idx_hbm, out_hbm, idx_v, buf_v, zbuf_v, acc_sh)


BN = 1000


def _mlp_body(x_ref, p0_ref, p1_ref, w1x_ref, w1e_ref, b1_ref, w2_ref,
              b2_ref, g_ref, bt_ref, o_ref):
    xb = x_ref[...]
    sb = p0_ref[...] + p1_ref[...]
    h = jnp.dot(xb, w1x_ref[...], preferred_element_type=jnp.float32)
    h = h + jnp.dot(sb, w1e_ref[...], preferred_element_type=jnp.float32)
    h = jnp.maximum(h + b1_ref[...], 0.0)
    o = jnp.dot(h, w2_ref[...], preferred_element_type=jnp.float32)
    o = o + b2_ref[...] + xb
    mu = jnp.mean(o, axis=-1, keepdims=True)
    d = o - mu
    var = jnp.mean(d * d, axis=-1, keepdims=True)
    o_ref[...] = d * lax.rsqrt(var + 1e-5) * g_ref[...] + bt_ref[...]


def _mlp(x2, p0, p1, w1x, w1e, b1, w2, b2, g, bt):
    full = pl.BlockSpec((H, H), lambda i: (0, 0))
    vec = pl.BlockSpec((1, H), lambda i: (0, 0))
    rows = pl.BlockSpec((BN, H), lambda i: (i, 0))
    return pl.pallas_call(
        _mlp_body,
        grid=(N // BN,),
        in_specs=[rows, rows, rows, full, full, vec, full, vec, vec, vec],
        out_specs=rows,
        out_shape=jax.ShapeDtypeStruct((N, H), jnp.float32),
    )(x2, p0, p1, w1x, w1e, b1, w2, b2, g, bt)


def kernel(x, edge_index, edge_attr, W1, b1, W2, b2, gamma, beta):
    row = edge_index[0, 0, :]
    ea = edge_attr[0]
    idx3 = row.reshape(NW, NCHUNK, CH)
    partial = _sc_scatter(ea, idx3)
    out = _mlp(x[0], partial[0], partial[1], W1[:H], W1[H:],
               b1.reshape(1, H), W2, b2.reshape(1, H),
               gamma.reshape(1, H), beta.reshape(1, H))
    return out[None]


# speedup vs baseline: 6.9876x; 1.5953x over previous
import functools

import jax
import jax.numpy as jnp
from jax import lax
from jax.experimental import pallas as pl
from jax.experimental.pallas import tpu as pltpu
from jax.experimental.pallas import tpu_sc as plsc

N = 10000
E = 320000
H = 128
NC = 2
NS = 16
NW = NC * NS
EPW = E // NW
CH = 80
NCHUNK = EPW // CH
NP = 10240
RPT = NP // NS


def _sc_scatter_body(ea_hbm, idx_hbm, out_hbm, idx_v, buf_v, acc_sh,
                     sem0, sem1):
    c = lax.axis_index("c")
    s = lax.axis_index("s")
    wid = s * NC + c

    def zstore(i, _):
        buf_v[0, i // 8, pl.ds((i % 8) * 16, 16)] = jnp.zeros((16,), jnp.float32)
        return 0
    lax.fori_loop(0, CH * 8, zstore, 0)
    for j in range(RPT // CH):
        pltpu.sync_copy(buf_v.at[0], acc_sh.at[pl.ds(s * RPT + j * CH, CH), :])
    plsc.subcore_barrier()

    pltpu.sync_copy(idx_hbm.at[wid], idx_v)
    base = wid * EPW

    def src(ci):
        return ea_hbm.at[pl.ds(base + ci * CH, CH), :]

    pltpu.async_copy(src(0), buf_v.at[0], sem0)

    def pair_body(i, _):
        cio = 2 * i
        pltpu.async_copy(src(cio + 1), buf_v.at[1], sem1)
        pltpu.make_async_copy(src(cio), buf_v.at[0], sem0).wait()
        pltpu.sync_copy(buf_v.at[0], acc_sh.at[idx_v.at[cio]], add=True)
        pltpu.async_copy(src(cio + 2), buf_v.at[0], sem0)
        pltpu.make_async_copy(src(cio + 1), buf_v.at[1], sem1).wait()
        pltpu.sync_copy(buf_v.at[1], acc_sh.at[idx_v.at[cio + 1]], add=True)
        return 0
    lax.fori_loop(0, (NCHUNK - 1) // 2, pair_body, 0)
    pltpu.make_async_copy(src(NCHUNK - 1), buf_v.at[0], sem0).wait()
    pltpu.sync_copy(buf_v.at[0], acc_sh.at[idx_v.at[NCHUNK - 1]], add=True)
    plsc.subcore_barrier()

    pltpu.sync_copy(acc_sh.at[pl.ds(s * RPT, RPT), :],
                    out_hbm.at[c, pl.ds(s * RPT, RPT), :])


@functools.partial(
    pl.kernel,
    out_type=jax.ShapeDtypeStruct((NC, NP, H), jnp.float32),
    mesh=plsc.VectorSubcoreMesh(core_axis_name="c", subcore_axis_name="s"),
    scratch_types=[
        pltpu.VMEM((NCHUNK, CH), jnp.int32),
        pltpu.VMEM((2, CH, H), jnp.float32),
        pltpu.VMEM_SHARED((NP, H), jnp.float32),
        pltpu.SemaphoreType.DMA,
        pltpu.SemaphoreType.DMA,
    ],
)
def _sc_scatter(ea_hbm, idx_hbm, out_hbm, idx_v, buf_v, acc_sh,
                sem0, sem1):
    _sc_scatter_body(ea_hbm, idx_hbm, out_hbm, idx_v, buf_v, acc_sh,
                     sem0, sem1)


BN = 1000


def _mlp_body(x_ref, p_ref, w1x_ref, w1e_ref, b1_ref, w2_ref,
              b2_ref, g_ref, bt_ref, o_ref):
    xb = x_ref[0]
    sb = p_ref[0] + p_ref[1]
    h = jnp.dot(xb, w1x_ref[...], preferred_element_type=jnp.float32)
    h = h + jnp.dot(sb, w1e_ref[...], preferred_element_type=jnp.float32)
    h = jnp.maximum(h + b1_ref[...], 0.0)
    o = jnp.dot(h, w2_ref[...], preferred_element_type=jnp.float32)
    o = o + b2_ref[...] + xb
    mu = jnp.mean(o, axis=-1, keepdims=True)
    d = o - mu
    var = jnp.mean(d * d, axis=-1, keepdims=True)
    o_ref[0] = d * lax.rsqrt(var + 1e-5) * g_ref[...] + bt_ref[...]


def _mlp(x, partial, w1x, w1e, b1, w2, b2, g, bt):
    full = pl.BlockSpec((H, H), lambda i: (0, 0))
    vec = pl.BlockSpec((1, H), lambda i: (0, 0))
    xrows = pl.BlockSpec((1, BN, H), lambda i: (0, i, 0))
    prows = pl.BlockSpec((2, BN, H), lambda i: (0, i, 0))
    return pl.pallas_call(
        _mlp_body,
        grid=(N // BN,),
        in_specs=[xrows, prows, full, full, vec, full, vec, vec, vec],
        out_specs=xrows,
        out_shape=jax.ShapeDtypeStruct((1, N, H), jnp.float32),
    )(x, partial, w1x, w1e, b1, w2, b2, g, bt)


def kernel(x, edge_index, edge_attr, W1, b1, W2, b2, gamma, beta):
    row = edge_index[0, 0, :]
    ea = edge_attr[0]
    idx3 = row.reshape(NW, NCHUNK, CH)
    partial = _sc_scatter(ea, idx3)
    return _mlp(x, partial, W1[:H], W1[H:],
                b1.reshape(1, H), W2, b2.reshape(1, H),
                gamma.reshape(1, H), beta.reshape(1, H))
